# baseline (speedup 1.0000x reference)
import functools

import jax
import jax.numpy as jnp
from jax import lax
from jax.experimental import pallas as pl
from jax.experimental.pallas import tpu as pltpu
from jax.experimental.pallas import tpu_sc as plsc

_H = 12
_T = 2048
_SPAN = 2048
_C = 2 * _SPAN + 1
_VW = 4096
_PAD = 128
_CT_W = 4352
_R = 1024
_SUB = 128

_NW = 32
_CHUNKS = _H * (_CT_W // 16)
_CPW = _CHUNKS // _NW
_TAB_FLAT = _C * _H


def _sc_table_rows_kernel(tab_hbm, ct_hbm, tab_v, out_v):
    wid = lax.axis_index("s") * 2 + lax.axis_index("c")
    pltpu.sync_copy(tab_hbm, tab_v)

    def body(k, _):
        cid = wid * _CPW + k
        h = cid // (_CT_W // 16)
        j = cid % (_CT_W // 16)
        m = j * 16 + lax.iota(jnp.int32, 16)
        x = m - _PAD
        valid = (x >= 0) & (x < _C)
        xi = jnp.clip(x, 0, _C - 1)
        vals = plsc.load_gather(tab_v, [xi * _H + h])
        vals = jnp.where(valid, vals, jnp.zeros((16,), jnp.float32))
        out_v[pl.ds(k * 16, 16)] = vals
        return _

    lax.fori_loop(0, _CPW, body, 0)
    pltpu.sync_copy(out_v, ct_hbm.at[pl.ds(wid * (_CPW * 16), _CPW * 16)])


def _sc_table_rows(table):
    k = pl.kernel(
        _sc_table_rows_kernel,
        out_type=jax.ShapeDtypeStruct((_H * _CT_W,), jnp.float32),
        mesh=plsc.VectorSubcoreMesh(core_axis_name="c", subcore_axis_name="s"),
        scratch_types=[
            pltpu.VMEM((_TAB_FLAT,), jnp.float32),
            pltpu.VMEM((_CPW * 16,), jnp.float32),
        ],
        compiler_params=pltpu.CompilerParams(needs_layout_passes=False),
    )
    return k(table.reshape(_TAB_FLAT)).reshape(_H, 1, _CT_W)


def _bias_add_kernel(tab_ref, attn_ref, out_ref, v_ref):
    q = pl.program_id(1)

    @pl.when(q == 0)
    def _build_skewed_table():
        for r in range(_SUB):
            v_ref[r, :] = tab_ref[0, 0, pl.ds(_PAD - r, _VW)]

    for s in range(_R // _SUB):
        row = _SUB * s
        start = _SUB * ((_SPAN // _SUB) - (_R // _SUB) * q - s)
        out_ref[0, pl.ds(row, _SUB), :] = (
            attn_ref[0, pl.ds(row, _SUB), :] + v_ref[:, pl.ds(start, _T)]
        )


def kernel(attn, table):
    h, b, t, l = attn.shape
    ct = _sc_table_rows(table)

    attn3 = attn.reshape(_H, t, l)
    nq = t // _R
    out3 = pl.pallas_call(
        _bias_add_kernel,
        grid=(_H, nq),
        in_specs=[
            pl.BlockSpec((1, 1, _CT_W), lambda hh, qq: (hh, 0, 0)),
            pl.BlockSpec((1, _R, _T), lambda hh, qq: (hh, qq, 0)),
        ],
        out_specs=pl.BlockSpec((1, _R, _T), lambda hh, qq: (hh, qq, 0)),
        out_shape=jax.ShapeDtypeStruct((_H, t, l), attn.dtype),
        scratch_shapes=[pltpu.VMEM((_SUB, _VW), jnp.float32)],
        compiler_params=pltpu.CompilerParams(
            dimension_semantics=("parallel", "arbitrary"),
        ),
    )(ct, attn3)
    return out3.reshape(attn.shape)


# speedup vs baseline: 1.2115x; 1.2115x over previous
import jax
import jax.numpy as jnp
from jax.experimental import pallas as pl
from jax.experimental.pallas import tpu as pltpu

_H = 12
_T = 2048
_SPAN = 2048
_VW = 4096
_PAD = 128
_CT_W = 4352
_R = 1024
_SUB = 128


def _bias_add_kernel(tab_ref, attn_ref, out_ref, v_ref):
    q = pl.program_id(1)

    @pl.when(q == 0)
    def _build_skewed_table():
        for r in range(_SUB):
            v_ref[r, :] = tab_ref[0, 0, pl.ds(_PAD - r, _VW)]

    for s in range(_R // _SUB):
        row = _SUB * s
        start = _SUB * ((_SPAN // _SUB) - (_R // _SUB) * q - s)
        out_ref[0, pl.ds(row, _SUB), :] = (
            attn_ref[0, pl.ds(row, _SUB), :] + v_ref[:, pl.ds(start, _T)]
        )


def kernel(attn, table):
    h, b, t, l = attn.shape
    ct = jnp.zeros((_H, 1, _CT_W), dtype=attn.dtype)
    ct = jax.lax.dynamic_update_slice(
        ct, table.T.reshape(_H, 1, 2 * _SPAN + 1), (0, 0, _PAD))

    attn3 = attn.reshape(_H, t, l)
    nq = t // _R
    out3 = pl.pallas_call(
        _bias_add_kernel,
        grid=(_H, nq),
        in_specs=[
            pl.BlockSpec((1, 1, _CT_W), lambda hh, qq: (hh, 0, 0)),
            pl.BlockSpec((1, _R, _T), lambda hh, qq: (hh, qq, 0)),
        ],
        out_specs=pl.BlockSpec((1, _R, _T), lambda hh, qq: (hh, qq, 0)),
        out_shape=jax.ShapeDtypeStruct((_H, t, l), attn.dtype),
        scratch_shapes=[pltpu.VMEM((_SUB, _VW), jnp.float32)],
        compiler_params=pltpu.CompilerParams(
            dimension_semantics=("parallel", "arbitrary"),
        ),
    )(ct, attn3)
    return out3.reshape(attn.shape)
